# baseline (speedup 1.0000x reference)
import functools

import jax
import jax.numpy as jnp
from jax import lax
from jax.experimental import pallas as pl
from jax.experimental.pallas import tpu as pltpu
from jax.experimental.pallas import tpu_sc as plsc

N_NODES = 10000
N_SUBCORES = 16
EDGES_PER_TILE = 10240
CHUNK = 128
BANK = 4
ACC_ROWS = 10112
ZROWS = ACC_ROWS // N_SUBCORES


def _accumulate(tab, srcs_t, dsts_t, acc, sb, db, gb, isem, gsem, n_chunks):
    n_banks = n_chunks // BANK
    assert n_banks % 2 == 0 and n_chunks % BANK == 0

    def fill(bank_idx, sbank, dbank):
        pltpu.async_copy(srcs_t.at[pl.ds(bank_idx * BANK, BANK)], sbank, isem)
        pltpu.async_copy(dsts_t.at[pl.ds(bank_idx * BANK, BANK)], dbank, isem)

    def fill_wait(sbank, dbank):
        pltpu.make_async_copy(srcs_t.at[pl.ds(0, BANK)], sbank, isem).wait()
        pltpu.make_async_copy(dsts_t.at[pl.ds(0, BANK)], dbank, isem).wait()

    def gather(sbank, t, g):
        pltpu.async_copy(tab.at[sbank.at[t]], gb[g], gsem[g])

    def gather_wait(g):
        pltpu.make_async_copy(tab.at[sb[0].at[0]], gb[g], gsem[g]).wait()

    fill(0, sb[0], db[0])
    fill_wait(sb[0], db[0])
    fill(1, sb[1], db[1])
    gather(sb[0], 0, 0)

    def body(m, carry):
        for half in range(2):
            for t in range(BANK):
                g = t % 2
                if t + 1 < BANK:
                    gather(sb[half], t + 1, 1 - g)
                elif half == 0:
                    fill_wait(sb[1], db[1])
                    gather(sb[1], 0, 1 - g)
                else:
                    @pl.when(2 * m + 2 < n_banks)
                    def _(g=g):
                        fill_wait(sb[0], db[0])
                        gather(sb[0], 0, 1 - g)
                gather_wait(g)
                pltpu.sync_copy(gb[g], acc.at[db[half].at[t]], add=True)
            @pl.when(2 * m + 2 + half < n_banks)
            def _(half=half):
                fill(2 * m + 2 + half, sb[half], db[half])
        return carry

    lax.fori_loop(0, n_banks // 2, body, 0)


def _make_segsum(width, tabs_per_core):
    mesh = plsc.VectorSubcoreMesh(core_axis_name="c", subcore_axis_name="s")
    n_tabs = 2 * tabs_per_core
    n_chunks = EDGES_PER_TILE // CHUNK

    @functools.partial(
        pl.kernel,
        out_type=[jax.ShapeDtypeStruct((ACC_ROWS, width), jnp.float32)
                  for _ in range(n_tabs)],
        mesh=mesh,
        compiler_params=pltpu.CompilerParams(use_tc_tiling_on_sc=False),
        scratch_types=[
            pltpu.VMEM((BANK, CHUNK), jnp.int32),
            pltpu.VMEM((BANK, CHUNK), jnp.int32),
            pltpu.VMEM((BANK, CHUNK), jnp.int32),
            pltpu.VMEM((BANK, CHUNK), jnp.int32),
            pltpu.VMEM((CHUNK, width), jnp.float32),
            pltpu.VMEM((CHUNK, width), jnp.float32),
            pltpu.VMEM_SHARED((ACC_ROWS, width), jnp.float32),
            pltpu.SemaphoreType.DMA,
            pltpu.SemaphoreType.DMA,
            pltpu.SemaphoreType.DMA,
        ],
    )
    def segsum(*args):
        tabs = args[:n_tabs]
        srcs, dsts, zeros = args[n_tabs:n_tabs + 3]
        outs = args[n_tabs + 3:2 * n_tabs + 3]
        (sb0, sb1, db0, db1, gb0, gb1, acc,
         isem, gsem0, gsem1) = args[2 * n_tabs + 3:]
        c = lax.axis_index("c")
        s = lax.axis_index("s")

        def one_pass(tab, out):
            pltpu.sync_copy(zeros, acc.at[pl.ds(s * ZROWS, ZROWS)])
            plsc.subcore_barrier()
            _accumulate(tab, srcs.at[s], dsts.at[s], acc, (sb0, sb1),
                        (db0, db1), (gb0, gb1), isem, (gsem0, gsem1),
                        n_chunks)
            plsc.subcore_barrier()
            pltpu.sync_copy(acc.at[pl.ds(s * ZROWS, ZROWS)],
                            out.at[pl.ds(s * ZROWS, ZROWS)])

        for t in range(tabs_per_core):
            @pl.when(c == 0)
            def _(t=t):
                one_pass(tabs[t], outs[t])

            @pl.when(c == 1)
            def _(t=t):
                one_pass(tabs[tabs_per_core + t], outs[tabs_per_core + t])
            if t + 1 < tabs_per_core:
                plsc.subcore_barrier()

    return segsum


def _xr_body(xr_ref, wr_ref, b_ref, o_ref):
    o_ref[...] = jnp.dot(xr_ref[...], wr_ref[...],
                         preferred_element_type=jnp.float32) + b_ref[...]


def _agg_body(agg_ref, d_ref, wl_ref, p_ref, o_ref, *, relu):
    inv = 1.0 / jnp.maximum(d_ref[...], 1.0)
    acc = jnp.dot(agg_ref[...] * inv, wl_ref[...],
                  preferred_element_type=jnp.float32) + p_ref[...]
    o_ref[...] = jnp.maximum(acc, 0.0) if relu else acc


def _dense_xr(xr, wr, bias, mb=1000):
    m, k = xr.shape
    n = wr.shape[1]
    return pl.pallas_call(
        _xr_body,
        grid=(m // mb,),
        in_specs=[
            pl.BlockSpec((mb, k), lambda i: (i, 0)),
            pl.BlockSpec((k, n), lambda i: (0, 0)),
            pl.BlockSpec((1, n), lambda i: (0, 0)),
        ],
        out_specs=pl.BlockSpec((mb, n), lambda i: (i, 0)),
        out_shape=jax.ShapeDtypeStruct((m, n), jnp.float32),
    )(xr, wr, bias)


def _dense_agg(agg, dcol, wl, p, relu, mb=1000):
    m, k = agg.shape
    n = wl.shape[1]
    return pl.pallas_call(
        functools.partial(_agg_body, relu=relu),
        grid=(m // mb,),
        in_specs=[
            pl.BlockSpec((mb, k), lambda i: (i, 0)),
            pl.BlockSpec((mb, 1), lambda i: (i, 0)),
            pl.BlockSpec((k, n), lambda i: (0, 0)),
            pl.BlockSpec((mb, n), lambda i: (i, 0)),
        ],
        out_specs=pl.BlockSpec((mb, n), lambda i: (i, 0)),
        out_shape=jax.ShapeDtypeStruct((m, n), jnp.float32),
    )(agg, dcol, wl, p)


def kernel(x, edge_index, W1_l, b1, W1_r, W2_l, b2, W2_r):
    src = edge_index[0].astype(jnp.int32)
    dst = edge_index[1].astype(jnp.int32)
    n_edges = src.shape[0]

    e_pad = N_SUBCORES * EDGES_PER_TILE - n_edges
    src_p = jnp.concatenate([src, jnp.zeros((e_pad,), jnp.int32)])
    dst_p = jnp.concatenate([dst, jnp.full((e_pad,), N_NODES, jnp.int32)])

    srcs = src_p.reshape(N_SUBCORES, EDGES_PER_TILE // CHUNK, CHUNK)
    dsts = dst_p.reshape(N_SUBCORES, EDGES_PER_TILE // CHUNK, CHUNK)

    tab0 = x[:, :144]
    tab1 = jnp.concatenate(
        [x[:, 144:], jnp.ones((N_NODES, 32), jnp.float32)], axis=1)
    z144 = jnp.zeros((ZROWS, 144), jnp.float32)
    agg_a, agg_b = _make_segsum(144, 1)(tab0, tab1, srcs, dsts, z144)
    p1 = _dense_xr(x, W1_r, b1.reshape(1, -1))
    agg1 = jnp.concatenate([agg_a[:N_NODES], agg_b[:N_NODES, :112]], axis=1)
    dcol = agg_b[:N_NODES, 112:113]
    h = _dense_agg(agg1, dcol, W1_l, p1, relu=True)

    z128 = jnp.zeros((ZROWS, 128), jnp.float32)
    a20, a21, a22, a23 = _make_segsum(128, 2)(
        h[:, 0:128], h[:, 128:256], h[:, 256:384], h[:, 384:512],
        srcs, dsts, z128)
    p2 = _dense_xr(h, W2_r, b2.reshape(1, -1))
    agg2 = jnp.concatenate(
        [a20[:N_NODES], a21[:N_NODES], a22[:N_NODES], a23[:N_NODES]], axis=1)
    out = _dense_agg(agg2, dcol, W2_l, p2, relu=False)
    return out


# speedup vs baseline: 1.1094x; 1.1094x over previous
import functools

import jax
import jax.numpy as jnp
from jax import lax
from jax.experimental import pallas as pl
from jax.experimental.pallas import tpu as pltpu
from jax.experimental.pallas import tpu_sc as plsc

N_NODES = 10000
N_SUBCORES = 16
EDGES_PER_TILE = 10112
CHUNK = 128
N_CHUNKS = EDGES_PER_TILE // CHUNK
WIDTH = 64
DEGW = 16
ACC_ROWS = 10112
ZROWS = ACC_ROWS // N_SUBCORES
TROWS = 632


def _make_segsum(tabs_per_core, with_deg):
    mesh = plsc.VectorSubcoreMesh(core_axis_name="c", subcore_axis_name="s")
    n_tabs = 2 * tabs_per_core
    n_in = n_tabs + 3 + (2 if with_deg else 0)
    n_out = n_tabs + (2 if with_deg else 0)

    scratch = [
        pltpu.VMEM((N_CHUNKS, CHUNK), jnp.int32),
        pltpu.VMEM((N_CHUNKS, CHUNK), jnp.int32),
        pltpu.VMEM((CHUNK, WIDTH), jnp.float32),
        pltpu.VMEM_SHARED((N_NODES, WIDTH), jnp.float32),
        pltpu.VMEM_SHARED((ACC_ROWS, WIDTH), jnp.float32),
        pltpu.SemaphoreType.DMA,
    ]
    if with_deg:
        scratch += [
            pltpu.VMEM((CHUNK, DEGW), jnp.float32),
            pltpu.VMEM_SHARED((ACC_ROWS, DEGW), jnp.float32),
        ]

    @functools.partial(
        pl.kernel,
        out_type=[jax.ShapeDtypeStruct(
            (ACC_ROWS, DEGW if with_deg and i >= n_tabs else WIDTH),
            jnp.float32) for i in range(n_out)],
        mesh=mesh,
        compiler_params=pltpu.CompilerParams(use_tc_tiling_on_sc=False),
        scratch_types=scratch,
    )
    def segsum(*args):
        tabs = args[:n_tabs]
        srcs, dsts, zeros = args[n_tabs:n_tabs + 3]
        if with_deg:
            ones, zeros_d = args[n_tabs + 3:n_in]
        outs = args[n_in:n_in + n_tabs]
        rest = args[n_in + n_tabs:]
        if with_deg:
            deg_outs = rest[:2]
            src_v, dst_v, gbuf, tab_s, acc, sem, ones_v, acc_d = rest[2:]
        else:
            src_v, dst_v, gbuf, tab_s, acc, sem = rest
        c = lax.axis_index("c")
        s = lax.axis_index("s")
        pltpu.sync_copy(srcs.at[s], src_v)
        pltpu.sync_copy(dsts.at[s], dst_v)

        def stage(tab):
            @pl.when(s < N_SUBCORES - 1)
            def _():
                pltpu.sync_copy(tab.at[pl.ds(s * TROWS, TROWS)],
                                tab_s.at[pl.ds(s * TROWS, TROWS)])

            @pl.when(s == N_SUBCORES - 1)
            def _():
                pltpu.sync_copy(
                    tab.at[pl.ds((N_SUBCORES - 1) * TROWS,
                                 N_NODES - (N_SUBCORES - 1) * TROWS)],
                    tab_s.at[pl.ds((N_SUBCORES - 1) * TROWS,
                                   N_NODES - (N_SUBCORES - 1) * TROWS)])

        def one_pass(tab, out):
            stage(tab)
            pltpu.sync_copy(zeros, acc.at[pl.ds(s * ZROWS, ZROWS)])
            plsc.subcore_barrier()

            def body(j, carry):
                pltpu.async_copy(tab_s.at[src_v.at[j]], gbuf, sem).wait()
                pltpu.sync_copy(gbuf, acc.at[dst_v.at[j]], add=True)
                return carry

            lax.fori_loop(0, N_CHUNKS, body, 0)
            plsc.subcore_barrier()
            pltpu.sync_copy(acc.at[pl.ds(s * ZROWS, ZROWS)],
                            out.at[pl.ds(s * ZROWS, ZROWS)])

        def deg_pass(lo, hi, out_d):
            pltpu.sync_copy(ones, ones_v)
            pltpu.sync_copy(zeros_d, acc_d.at[pl.ds(s * ZROWS, ZROWS)])
            plsc.subcore_barrier()

            def body(j, carry):
                pltpu.sync_copy(ones_v, acc_d.at[dst_v.at[j]], add=True)
                return carry

            lax.fori_loop(lo, hi, body, 0)
            plsc.subcore_barrier()
            pltpu.sync_copy(acc_d.at[pl.ds(s * ZROWS, ZROWS)],
                            out_d.at[pl.ds(s * ZROWS, ZROWS)])

        for t in range(tabs_per_core):
            @pl.when(c == 0)
            def _(t=t):
                one_pass(tabs[t], outs[t])

            @pl.when(c == 1)
            def _(t=t):
                one_pass(tabs[tabs_per_core + t], outs[tabs_per_core + t])
            plsc.subcore_barrier()

        if with_deg:
            half = N_CHUNKS // 2

            @pl.when(c == 0)
            def _():
                deg_pass(0, half, deg_outs[0])

            @pl.when(c == 1)
            def _():
                deg_pass(half, N_CHUNKS, deg_outs[1])

    return segsum


def _xr_body(xr_ref, wr_ref, b_ref, o_ref):
    o_ref[...] = jnp.dot(xr_ref[...], wr_ref[...],
                         preferred_element_type=jnp.float32) + b_ref[...]


def _agg_body(agg_ref, d0_ref, d1_ref, wl_ref, p_ref, o_ref, *, relu):
    inv = 1.0 / jnp.maximum(d0_ref[...] + d1_ref[...], 1.0)
    acc = jnp.dot(agg_ref[...] * inv, wl_ref[...],
                  preferred_element_type=jnp.float32) + p_ref[...]
    o_ref[...] = jnp.maximum(acc, 0.0) if relu else acc


def _dense_xr(xr, wr, bias, mb=1000):
    m, k = xr.shape
    n = wr.shape[1]
    return pl.pallas_call(
        _xr_body,
        grid=(m // mb,),
        in_specs=[
            pl.BlockSpec((mb, k), lambda i: (i, 0)),
            pl.BlockSpec((k, n), lambda i: (0, 0)),
            pl.BlockSpec((1, n), lambda i: (0, 0)),
        ],
        out_specs=pl.BlockSpec((mb, n), lambda i: (i, 0)),
        out_shape=jax.ShapeDtypeStruct((m, n), jnp.float32),
    )(xr, wr, bias)


def _dense_agg(agg, d0, d1, wl, p, relu, mb=1000):
    m, k = agg.shape
    n = wl.shape[1]
    return pl.pallas_call(
        functools.partial(_agg_body, relu=relu),
        grid=(m // mb,),
        in_specs=[
            pl.BlockSpec((mb, k), lambda i: (i, 0)),
            pl.BlockSpec((mb, 1), lambda i: (i, 0)),
            pl.BlockSpec((mb, 1), lambda i: (i, 0)),
            pl.BlockSpec((k, n), lambda i: (0, 0)),
            pl.BlockSpec((mb, n), lambda i: (i, 0)),
        ],
        out_specs=pl.BlockSpec((mb, n), lambda i: (i, 0)),
        out_shape=jax.ShapeDtypeStruct((m, n), jnp.float32),
    )(agg, d0, d1, wl, p)


def kernel(x, edge_index, W1_l, b1, W1_r, W2_l, b2, W2_r):
    src = edge_index[0].astype(jnp.int32)
    dst = edge_index[1].astype(jnp.int32)
    n_edges = src.shape[0]

    e_pad = N_SUBCORES * EDGES_PER_TILE - n_edges
    src_p = jnp.concatenate([src, jnp.zeros((e_pad,), jnp.int32)])
    dst_p = jnp.concatenate([dst, jnp.full((e_pad,), N_NODES, jnp.int32)])
    srcs = src_p.reshape(N_SUBCORES, N_CHUNKS, CHUNK)
    dsts = dst_p.reshape(N_SUBCORES, N_CHUNKS, CHUNK)

    zeros = jnp.zeros((ZROWS, WIDTH), jnp.float32)
    zeros_d = jnp.zeros((ZROWS, DEGW), jnp.float32)
    ones = jnp.ones((CHUNK, DEGW), jnp.float32)

    tabs1 = [x[:, k * WIDTH:(k + 1) * WIDTH] for k in range(4)]
    o0, o1, o2, o3, d0, d1 = _make_segsum(2, True)(
        *tabs1, srcs, dsts, zeros, ones, zeros_d)
    p1 = _dense_xr(x, W1_r, b1.reshape(1, -1))
    agg1 = jnp.concatenate(
        [o0[:N_NODES], o1[:N_NODES], o2[:N_NODES], o3[:N_NODES]], axis=1)
    dc0 = d0[:N_NODES, 0:1]
    dc1 = d1[:N_NODES, 0:1]
    h = _dense_agg(agg1, dc0, dc1, W1_l, p1, relu=True)

    tabs2 = [h[:, k * WIDTH:(k + 1) * WIDTH] for k in range(8)]
    a2 = _make_segsum(4, False)(*tabs2, srcs, dsts, zeros)
    p2 = _dense_xr(h, W2_r, b2.reshape(1, -1))
    agg2 = jnp.concatenate([a[:N_NODES] for a in a2], axis=1)
    out = _dense_agg(agg2, dc0, dc1, W2_l, p2, relu=False)
    return out


# speedup vs baseline: 1.2420x; 1.1195x over previous
import functools

import jax
import jax.numpy as jnp
from jax import lax
from jax.experimental import pallas as pl
from jax.experimental.pallas import tpu as pltpu
from jax.experimental.pallas import tpu_sc as plsc

N_NODES = 10000
N_SUBCORES = 16
EDGES_PER_TILE = 10112
CHUNK = 128
N_CHUNKS = EDGES_PER_TILE // CHUNK
WIDTH = 64
DEGW = 16
ACC_ROWS = 10112
ZROWS = ACC_ROWS // N_SUBCORES
TROWS = 632


def _make_segsum(tabs_per_core, with_deg):
    mesh = plsc.VectorSubcoreMesh(core_axis_name="c", subcore_axis_name="s")
    n_tabs = 2 * tabs_per_core
    n_in = n_tabs + 3 + (2 if with_deg else 0)
    n_out = n_tabs + (2 if with_deg else 0)

    scratch = [
        pltpu.VMEM((N_CHUNKS, CHUNK), jnp.int32),
        pltpu.VMEM((N_CHUNKS, CHUNK), jnp.int32),
        pltpu.VMEM((CHUNK, WIDTH), jnp.float32),
        pltpu.VMEM_SHARED((N_NODES, WIDTH), jnp.float32),
        pltpu.VMEM_SHARED((ACC_ROWS, WIDTH), jnp.float32),
        pltpu.SemaphoreType.DMA,
    ]
    if with_deg:
        scratch += [
            pltpu.VMEM((CHUNK, DEGW), jnp.float32),
            pltpu.VMEM_SHARED((ACC_ROWS, DEGW), jnp.float32),
        ]

    @functools.partial(
        pl.kernel,
        out_type=[jax.ShapeDtypeStruct(
            (ACC_ROWS, DEGW if with_deg and i >= n_tabs else WIDTH),
            jnp.float32) for i in range(n_out)],
        mesh=mesh,
        compiler_params=pltpu.CompilerParams(use_tc_tiling_on_sc=False),
        scratch_types=scratch,
    )
    def segsum(*args):
        tabs = args[:n_tabs]
        srcs, dsts, zeros = args[n_tabs:n_tabs + 3]
        if with_deg:
            ones, zeros_d = args[n_tabs + 3:n_in]
        outs = args[n_in:n_in + n_tabs]
        rest = args[n_in + n_tabs:]
        if with_deg:
            deg_outs = rest[:2]
            src_v, dst_v, gbuf, tab_s, acc, sem, ones_v, acc_d = rest[2:]
        else:
            src_v, dst_v, gbuf, tab_s, acc, sem = rest
        c = lax.axis_index("c")
        s = lax.axis_index("s")
        pltpu.sync_copy(srcs.at[s], src_v)
        pltpu.sync_copy(dsts.at[s], dst_v)

        def stage(tab):
            @pl.when(s < N_SUBCORES - 1)
            def _():
                pltpu.sync_copy(tab.at[pl.ds(s * TROWS, TROWS)],
                                tab_s.at[pl.ds(s * TROWS, TROWS)])

            @pl.when(s == N_SUBCORES - 1)
            def _():
                pltpu.sync_copy(
                    tab.at[pl.ds((N_SUBCORES - 1) * TROWS,
                                 N_NODES - (N_SUBCORES - 1) * TROWS)],
                    tab_s.at[pl.ds((N_SUBCORES - 1) * TROWS,
                                   N_NODES - (N_SUBCORES - 1) * TROWS)])

        def one_pass(tab, out):
            stage(tab)
            pltpu.sync_copy(zeros, acc.at[pl.ds(s * ZROWS, ZROWS)])
            plsc.subcore_barrier()

            def body(j, carry):
                pltpu.async_copy(tab_s.at[src_v.at[j]], gbuf, sem).wait()
                pltpu.sync_copy(gbuf, acc.at[dst_v.at[j]], add=True)
                return carry

            lax.fori_loop(0, N_CHUNKS, body, 0)
            plsc.subcore_barrier()
            pltpu.sync_copy(acc.at[pl.ds(s * ZROWS, ZROWS)],
                            out.at[pl.ds(s * ZROWS, ZROWS)])

        def deg_pass(lo, hi, out_d):
            pltpu.sync_copy(ones, ones_v)
            pltpu.sync_copy(zeros_d, acc_d.at[pl.ds(s * ZROWS, ZROWS)])
            plsc.subcore_barrier()

            def body(j, carry):
                pltpu.sync_copy(ones_v, acc_d.at[dst_v.at[j]], add=True)
                return carry

            lax.fori_loop(lo, hi, body, 0)
            plsc.subcore_barrier()
            pltpu.sync_copy(acc_d.at[pl.ds(s * ZROWS, ZROWS)],
                            out_d.at[pl.ds(s * ZROWS, ZROWS)])

        for t in range(tabs_per_core):
            @pl.when(c == 0)
            def _(t=t):
                one_pass(tabs[t], outs[t])

            @pl.when(c == 1)
            def _(t=t):
                one_pass(tabs[tabs_per_core + t], outs[tabs_per_core + t])
            plsc.subcore_barrier()

        if with_deg:
            half = N_CHUNKS // 2

            @pl.when(c == 0)
            def _():
                deg_pass(0, half, deg_outs[0])

            @pl.when(c == 1)
            def _():
                deg_pass(half, N_CHUNKS, deg_outs[1])

    return segsum


def _dense_body(n_agg, n_xr, n_out, relu, *refs):
    aggs = refs[:n_agg]
    d0_ref, d1_ref = refs[n_agg:n_agg + 2]
    xrs = refs[n_agg + 2:n_agg + 2 + n_xr]
    wl_ref, wr_ref, b_ref = refs[n_agg + 2 + n_xr:n_agg + 5 + n_xr]
    outs = refs[n_agg + 5 + n_xr:]
    deg = d0_ref[:, 0:1] + d1_ref[:, 0:1]
    inv = 1.0 / jnp.maximum(deg, 1.0)
    mean = jnp.concatenate([a[...] * inv for a in aggs], axis=1)
    xcat = (xrs[0][...] if n_xr == 1 else
            jnp.concatenate([r[...] for r in xrs], axis=1))
    acc = jnp.dot(mean, wl_ref[...], preferred_element_type=jnp.float32)
    acc = acc + jnp.dot(xcat, wr_ref[...],
                        preferred_element_type=jnp.float32)
    acc = acc + b_ref[...]
    if relu:
        acc = jnp.maximum(acc, 0.0)
    if n_out == 1:
        outs[0][...] = acc
    else:
        w = acc.shape[1] // n_out
        for k in range(n_out):
            outs[k][...] = acc[:, k * w:(k + 1) * w]


def _dense_layer(aggs, d0, d1, xrs, wl, wr, bias, relu, n_out, mb=1000):
    m = xrs[0].shape[0]
    k = wl.shape[0]
    n = wl.shape[1]
    wa = aggs[0].shape[1]
    wx = xrs[0].shape[1]
    out_shape = [jax.ShapeDtypeStruct((m, n // n_out), jnp.float32)
                 for _ in range(n_out)]
    res = pl.pallas_call(
        functools.partial(_dense_body, len(aggs), len(xrs), n_out, relu),
        grid=(m // mb,),
        in_specs=(
            [pl.BlockSpec((mb, wa), lambda i: (i, 0)) for _ in aggs]
            + [pl.BlockSpec((mb, DEGW), lambda i: (i, 0))] * 2
            + [pl.BlockSpec((mb, wx), lambda i: (i, 0)) for _ in xrs]
            + [pl.BlockSpec((k, n), lambda i: (0, 0)),
               pl.BlockSpec((k, n), lambda i: (0, 0)),
               pl.BlockSpec((1, n), lambda i: (0, 0))]
        ),
        out_specs=[pl.BlockSpec((mb, n // n_out), lambda i: (i, 0))
                   for _ in range(n_out)],
        out_shape=out_shape,
    )(*aggs, d0, d1, *xrs, wl, wr, bias)
    return res


def kernel(x, edge_index, W1_l, b1, W1_r, W2_l, b2, W2_r):
    src = edge_index[0].astype(jnp.int32)
    dst = edge_index[1].astype(jnp.int32)
    n_edges = src.shape[0]

    e_pad = N_SUBCORES * EDGES_PER_TILE - n_edges
    src_p = jnp.concatenate([src, jnp.zeros((e_pad,), jnp.int32)])
    dst_p = jnp.concatenate([dst, jnp.full((e_pad,), N_NODES, jnp.int32)])
    srcs = src_p.reshape(N_SUBCORES, N_CHUNKS, CHUNK)
    dsts = dst_p.reshape(N_SUBCORES, N_CHUNKS, CHUNK)

    zeros = jnp.zeros((ZROWS, WIDTH), jnp.float32)
    zeros_d = jnp.zeros((ZROWS, DEGW), jnp.float32)
    ones = jnp.ones((CHUNK, DEGW), jnp.float32)

    tabs1 = [x[:, k * WIDTH:(k + 1) * WIDTH] for k in range(4)]
    o0, o1, o2, o3, d0, d1 = _make_segsum(2, True)(
        *tabs1, srcs, dsts, zeros, ones, zeros_d)
    hs = _dense_layer([o0, o1, o2, o3], d0, d1, [x], W1_l, W1_r,
                      b1.reshape(1, -1), relu=True, n_out=8)

    a2 = _make_segsum(4, False)(*hs, srcs, dsts, zeros)
    out, = _dense_layer(list(a2), d0, d1, hs, W2_l, W2_r, b2.reshape(1, -1),
                        relu=False, n_out=1)
    return out


# speedup vs baseline: 1.5701x; 1.2642x over previous
import functools

import jax
import jax.numpy as jnp
from jax import lax
from jax.experimental import pallas as pl
from jax.experimental.pallas import tpu as pltpu
from jax.experimental.pallas import tpu_sc as plsc

N_NODES = 10000
N_SUBCORES = 16
EDGES_PER_TILE = 10112
CHUNK = 128
N_CHUNKS = EDGES_PER_TILE // CHUNK
WIDTH = 64
DEGW = 16
ACC_ROWS = 10112
ZROWS = ACC_ROWS // N_SUBCORES
TROWS = 632


def _make_segsum(tabs_per_core, with_deg):
    mesh = plsc.VectorSubcoreMesh(core_axis_name="c", subcore_axis_name="s")
    n_tabs = 2 * tabs_per_core
    n_in = n_tabs + 3 + (2 if with_deg else 0)
    n_out = n_tabs + (2 if with_deg else 0)

    scratch = [
        pltpu.VMEM((N_CHUNKS, CHUNK), jnp.int32),
        pltpu.VMEM((N_CHUNKS, CHUNK), jnp.int32),
        pltpu.VMEM((CHUNK, WIDTH), jnp.float32),
        pltpu.VMEM((CHUNK, WIDTH), jnp.float32),
        pltpu.VMEM_SHARED((N_NODES, WIDTH), jnp.float32),
        pltpu.VMEM_SHARED((ACC_ROWS, WIDTH), jnp.float32),
        pltpu.SemaphoreType.DMA,
        pltpu.SemaphoreType.DMA,
    ]
    if with_deg:
        scratch += [
            pltpu.VMEM((CHUNK, DEGW), jnp.float32),
            pltpu.VMEM_SHARED((ACC_ROWS, DEGW), jnp.float32),
        ]

    @functools.partial(
        pl.kernel,
        out_type=[jax.ShapeDtypeStruct(
            (ACC_ROWS, DEGW if with_deg and i >= n_tabs else WIDTH),
            jnp.float32) for i in range(n_out)],
        mesh=mesh,
        compiler_params=pltpu.CompilerParams(use_tc_tiling_on_sc=False),
        scratch_types=scratch,
    )
    def segsum(*args):
        tabs = args[:n_tabs]
        srcs, dsts, zeros = args[n_tabs:n_tabs + 3]
        if with_deg:
            ones, zeros_d = args[n_tabs + 3:n_in]
        outs = args[n_in:n_in + n_tabs]
        rest = args[n_in + n_tabs:]
        if with_deg:
            deg_outs = rest[:2]
            (src_v, dst_v, gb0, gb1, tab_s, acc, sem0, sem1,
             ones_v, acc_d) = rest[2:]
        else:
            src_v, dst_v, gb0, gb1, tab_s, acc, sem0, sem1 = rest
        c = lax.axis_index("c")
        s = lax.axis_index("s")
        pltpu.sync_copy(srcs.at[s], src_v)
        pltpu.sync_copy(dsts.at[s], dst_v)

        def stage(tab):
            @pl.when(s < N_SUBCORES - 1)
            def _():
                pltpu.sync_copy(tab.at[pl.ds(s * TROWS, TROWS)],
                                tab_s.at[pl.ds(s * TROWS, TROWS)])

            @pl.when(s == N_SUBCORES - 1)
            def _():
                pltpu.sync_copy(
                    tab.at[pl.ds((N_SUBCORES - 1) * TROWS,
                                 N_NODES - (N_SUBCORES - 1) * TROWS)],
                    tab_s.at[pl.ds((N_SUBCORES - 1) * TROWS,
                                   N_NODES - (N_SUBCORES - 1) * TROWS)])

        def one_pass(tab, out):
            stage(tab)
            pltpu.sync_copy(zeros, acc.at[pl.ds(s * ZROWS, ZROWS)])
            plsc.subcore_barrier()

            def gather(j, gb, sem):
                pltpu.async_copy(tab_s.at[src_v.at[j]], gb, sem)

            def gwait(gb, sem):
                pltpu.make_async_copy(tab_s.at[src_v.at[0]], gb, sem).wait()

            gather(0, gb0, sem0)

            def body(i, carry):
                j = 2 * i
                gather(j + 1, gb1, sem1)
                gwait(gb0, sem0)
                pltpu.sync_copy(gb0, acc.at[dst_v.at[j]], add=True)
                gather(j + 2, gb0, sem0)
                gwait(gb1, sem1)
                pltpu.sync_copy(gb1, acc.at[dst_v.at[j + 1]], add=True)
                return carry

            lax.fori_loop(0, N_CHUNKS // 2, body, 0)
            gwait(gb0, sem0)
            pltpu.sync_copy(gb0, acc.at[dst_v.at[N_CHUNKS - 1]], add=True)
            plsc.subcore_barrier()
            pltpu.sync_copy(acc.at[pl.ds(s * ZROWS, ZROWS)],
                            out.at[pl.ds(s * ZROWS, ZROWS)])

        def deg_pass(lo, hi, out_d):
            pltpu.sync_copy(ones, ones_v)
            pltpu.sync_copy(zeros_d, acc_d.at[pl.ds(s * ZROWS, ZROWS)])
            plsc.subcore_barrier()

            def body(j, carry):
                pltpu.sync_copy(ones_v, acc_d.at[dst_v.at[j]], add=True)
                return carry

            lax.fori_loop(lo, hi, body, 0)
            plsc.subcore_barrier()
            pltpu.sync_copy(acc_d.at[pl.ds(s * ZROWS, ZROWS)],
                            out_d.at[pl.ds(s * ZROWS, ZROWS)])

        for t in range(tabs_per_core):
            @pl.when(c == 0)
            def _(t=t):
                one_pass(tabs[t], outs[t])

            @pl.when(c == 1)
            def _(t=t):
                one_pass(tabs[tabs_per_core + t], outs[tabs_per_core + t])
            plsc.subcore_barrier()

        if with_deg:
            half = N_CHUNKS // 2

            @pl.when(c == 0)
            def _():
                deg_pass(0, half, deg_outs[0])

            @pl.when(c == 1)
            def _():
                deg_pass(half, N_CHUNKS, deg_outs[1])

    return segsum


def _dense_body(n_agg, n_xr, n_out, relu, *refs):
    aggs = refs[:n_agg]
    d0_ref, d1_ref = refs[n_agg:n_agg + 2]
    xrs = refs[n_agg + 2:n_agg + 2 + n_xr]
    wl_ref, wr_ref, b_ref = refs[n_agg + 2 + n_xr:n_agg + 5 + n_xr]
    outs = refs[n_agg + 5 + n_xr:]
    deg = d0_ref[:, 0:1] + d1_ref[:, 0:1]
    inv = 1.0 / jnp.maximum(deg, 1.0)
    mean = jnp.concatenate([a[...] * inv for a in aggs], axis=1)
    xcat = (xrs[0][...] if n_xr == 1 else
            jnp.concatenate([r[...] for r in xrs], axis=1))
    acc = jnp.dot(mean, wl_ref[...], preferred_element_type=jnp.float32)
    acc = acc + jnp.dot(xcat, wr_ref[...],
                        preferred_element_type=jnp.float32)
    acc = acc + b_ref[...]
    if relu:
        acc = jnp.maximum(acc, 0.0)
    if n_out == 1:
        outs[0][...] = acc
    else:
        w = acc.shape[1] // n_out
        for k in range(n_out):
            outs[k][...] = acc[:, k * w:(k + 1) * w]


def _dense_layer(aggs, d0, d1, xrs, wl, wr, bias, relu, n_out, mb=1000):
    m = xrs[0].shape[0]
    k = wl.shape[0]
    n = wl.shape[1]
    wa = aggs[0].shape[1]
    wx = xrs[0].shape[1]
    out_shape = [jax.ShapeDtypeStruct((m, n // n_out), jnp.float32)
                 for _ in range(n_out)]
    res = pl.pallas_call(
        functools.partial(_dense_body, len(aggs), len(xrs), n_out, relu),
        grid=(m // mb,),
        in_specs=(
            [pl.BlockSpec((mb, wa), lambda i: (i, 0)) for _ in aggs]
            + [pl.BlockSpec((mb, DEGW), lambda i: (i, 0))] * 2
            + [pl.BlockSpec((mb, wx), lambda i: (i, 0)) for _ in xrs]
            + [pl.BlockSpec((k, n), lambda i: (0, 0)),
               pl.BlockSpec((k, n), lambda i: (0, 0)),
               pl.BlockSpec((1, n), lambda i: (0, 0))]
        ),
        out_specs=[pl.BlockSpec((mb, n // n_out), lambda i: (i, 0))
                   for _ in range(n_out)],
        out_shape=out_shape,
    )(*aggs, d0, d1, *xrs, wl, wr, bias)
    return res


def kernel(x, edge_index, W1_l, b1, W1_r, W2_l, b2, W2_r):
    src = edge_index[0].astype(jnp.int32)
    dst = edge_index[1].astype(jnp.int32)
    n_edges = src.shape[0]

    e_pad = N_SUBCORES * EDGES_PER_TILE - n_edges
    src_p = jnp.concatenate([src, jnp.zeros((e_pad,), jnp.int32)])
    dst_p = jnp.concatenate([dst, jnp.full((e_pad,), N_NODES, jnp.int32)])
    srcs = src_p.reshape(N_SUBCORES, N_CHUNKS, CHUNK)
    dsts = dst_p.reshape(N_SUBCORES, N_CHUNKS, CHUNK)

    zeros = jnp.zeros((ZROWS, WIDTH), jnp.float32)
    zeros_d = jnp.zeros((ZROWS, DEGW), jnp.float32)
    ones = jnp.ones((CHUNK, DEGW), jnp.float32)

    tabs1 = [x[:, k * WIDTH:(k + 1) * WIDTH] for k in range(4)]
    o0, o1, o2, o3, d0, d1 = _make_segsum(2, True)(
        *tabs1, srcs, dsts, zeros, ones, zeros_d)
    hs = _dense_layer([o0, o1, o2, o3], d0, d1, [x], W1_l, W1_r,
                      b1.reshape(1, -1), relu=True, n_out=8)

    a2 = _make_segsum(4, False)(*hs, srcs, dsts, zeros)
    out, = _dense_layer(list(a2), d0, d1, hs, W2_l, W2_r, b2.reshape(1, -1),
                        relu=False, n_out=1)
    return out


# speedup vs baseline: 1.7076x; 1.0876x over previous
import functools

import jax
import jax.numpy as jnp
from jax import lax
from jax.experimental import pallas as pl
from jax.experimental.pallas import tpu as pltpu
from jax.experimental.pallas import tpu_sc as plsc

N_NODES = 10000
N_SUBCORES = 16
EDGES_PER_TILE = 10112
CHUNK = 128
N_CHUNKS = EDGES_PER_TILE // CHUNK
WIDTH = 64
DEGW = 16
ACC_ROWS = 10112
ZROWS = ACC_ROWS // N_SUBCORES
TROWS = 632


def _make_segsum(tabs_per_core, with_deg, nbuf=2):
    mesh = plsc.VectorSubcoreMesh(core_axis_name="c", subcore_axis_name="s")
    n_tabs = 2 * tabs_per_core
    n_in = n_tabs + 3 + (2 if with_deg else 0)
    n_out = n_tabs + (2 if with_deg else 0)

    scratch = (
        [pltpu.VMEM((N_CHUNKS, CHUNK), jnp.int32),
         pltpu.VMEM((N_CHUNKS, CHUNK), jnp.int32)]
        + [pltpu.VMEM((CHUNK, WIDTH), jnp.float32)] * nbuf
        + [pltpu.VMEM_SHARED((N_NODES, WIDTH), jnp.float32),
           pltpu.VMEM_SHARED((ACC_ROWS, WIDTH), jnp.float32)]
        + [pltpu.SemaphoreType.DMA] * (2 * nbuf)
    )
    if with_deg:
        scratch += [
            pltpu.VMEM((CHUNK, DEGW), jnp.float32),
            pltpu.VMEM_SHARED((ACC_ROWS, DEGW), jnp.float32),
        ]

    @functools.partial(
        pl.kernel,
        out_type=[jax.ShapeDtypeStruct(
            (ACC_ROWS, DEGW if with_deg and i >= n_tabs else WIDTH),
            jnp.float32) for i in range(n_out)],
        mesh=mesh,
        compiler_params=pltpu.CompilerParams(use_tc_tiling_on_sc=False),
        scratch_types=scratch,
    )
    def segsum(*args):
        tabs = args[:n_tabs]
        srcs, dsts, zeros = args[n_tabs:n_tabs + 3]
        if with_deg:
            ones, zeros_d = args[n_tabs + 3:n_in]
        outs = args[n_in:n_in + n_tabs]
        rest = args[n_in + n_tabs:]
        if with_deg:
            deg_outs = rest[:2]
            rest = rest[2:]
        src_v, dst_v = rest[:2]
        gb = rest[2:2 + nbuf]
        tab_s, acc = rest[2 + nbuf:4 + nbuf]
        gsem = rest[4 + nbuf:4 + 2 * nbuf]
        ssem = rest[4 + 2 * nbuf:4 + 3 * nbuf]
        if with_deg:
            ones_v, acc_d = rest[4 + 3 * nbuf:]
        c = lax.axis_index("c")
        s = lax.axis_index("s")
        pltpu.sync_copy(srcs.at[s], src_v)
        pltpu.sync_copy(dsts.at[s], dst_v)

        def stage(tab):
            @pl.when(s < N_SUBCORES - 1)
            def _():
                pltpu.sync_copy(tab.at[pl.ds(s * TROWS, TROWS)],
                                tab_s.at[pl.ds(s * TROWS, TROWS)])

            @pl.when(s == N_SUBCORES - 1)
            def _():
                pltpu.sync_copy(
                    tab.at[pl.ds((N_SUBCORES - 1) * TROWS,
                                 N_NODES - (N_SUBCORES - 1) * TROWS)],
                    tab_s.at[pl.ds((N_SUBCORES - 1) * TROWS,
                                   N_NODES - (N_SUBCORES - 1) * TROWS)])

        def gather(j, b):
            pltpu.async_copy(tab_s.at[src_v.at[j]], gb[b], gsem[b])

        def gwait(b):
            pltpu.make_async_copy(tab_s.at[src_v.at[0]], gb[b],
                                  gsem[b]).wait()

        def swait(b):
            pltpu.make_async_copy(gb[b], acc.at[dst_v.at[0]],
                                  ssem[b]).wait()

        def one_pass(tab, out):
            stage(tab)
            pltpu.sync_copy(zeros, acc.at[pl.ds(s * ZROWS, ZROWS)])
            plsc.subcore_barrier()

            if nbuf == 2:
                gather(0, 0)

                def body(i, carry):
                    j = 2 * i
                    gather(j + 1, 1)
                    gwait(0)
                    pltpu.sync_copy(gb[0], acc.at[dst_v.at[j]], add=True)
                    gather(j + 2, 0)
                    gwait(1)
                    pltpu.sync_copy(gb[1], acc.at[dst_v.at[j + 1]], add=True)
                    return carry

                lax.fori_loop(0, N_CHUNKS // 2, body, 0)
                gwait(0)
                pltpu.sync_copy(gb[0], acc.at[dst_v.at[N_CHUNKS - 1]],
                                add=True)
            else:
                gather(0, 0)
                gather(1, 1)

                def body(i, carry):
                    for k in range(3):
                        j = 3 * i + k
                        gwait(k)
                        pltpu.async_copy(gb[k], acc.at[dst_v.at[j]],
                                         ssem[k], add=True)

                        @pl.when(j >= 1)
                        def _(k=k):
                            swait((k + 2) % 3)

                        @pl.when(j + 2 < N_CHUNKS)
                        def _(j=j, k=k):
                            gather(j + 2, (k + 2) % 3)
                    return carry

                lax.fori_loop(0, N_CHUNKS // 3, body, 0)
                gwait(0)
                pltpu.async_copy(gb[0], acc.at[dst_v.at[N_CHUNKS - 1]],
                                 ssem[0], add=True)
                swait(2)
                swait(0)
            plsc.subcore_barrier()
            pltpu.sync_copy(acc.at[pl.ds(s * ZROWS, ZROWS)],
                            out.at[pl.ds(s * ZROWS, ZROWS)])

        def deg_pass(lo, hi, out_d):
            pltpu.sync_copy(ones, ones_v)
            pltpu.sync_copy(zeros_d, acc_d.at[pl.ds(s * ZROWS, ZROWS)])
            plsc.subcore_barrier()

            def body(j, carry):
                pltpu.sync_copy(ones_v, acc_d.at[dst_v.at[j]], add=True)
                return carry

            lax.fori_loop(lo, hi, body, 0)
            plsc.subcore_barrier()
            pltpu.sync_copy(acc_d.at[pl.ds(s * ZROWS, ZROWS)],
                            out_d.at[pl.ds(s * ZROWS, ZROWS)])

        for t in range(tabs_per_core):
            @pl.when(c == 0)
            def _(t=t):
                one_pass(tabs[t], outs[t])

            @pl.when(c == 1)
            def _(t=t):
                one_pass(tabs[tabs_per_core + t], outs[tabs_per_core + t])
            plsc.subcore_barrier()

        if with_deg:
            half = N_CHUNKS // 2

            @pl.when(c == 0)
            def _():
                deg_pass(0, half, deg_outs[0])

            @pl.when(c == 1)
            def _():
                deg_pass(half, N_CHUNKS, deg_outs[1])

    return segsum


def _dense_body(n_agg, n_xr, n_out, relu, *refs):
    aggs = refs[:n_agg]
    d0_ref, d1_ref = refs[n_agg:n_agg + 2]
    xrs = refs[n_agg + 2:n_agg + 2 + n_xr]
    wl_ref, wr_ref, b_ref = refs[n_agg + 2 + n_xr:n_agg + 5 + n_xr]
    outs = refs[n_agg + 5 + n_xr:]
    deg = d0_ref[:, 0:1] + d1_ref[:, 0:1]
    inv = 1.0 / jnp.maximum(deg, 1.0)
    mean = jnp.concatenate([a[...] * inv for a in aggs], axis=1)
    xcat = (xrs[0][...] if n_xr == 1 else
            jnp.concatenate([r[...] for r in xrs], axis=1))
    acc = jnp.dot(mean, wl_ref[...], preferred_element_type=jnp.float32)
    acc = acc + jnp.dot(xcat, wr_ref[...],
                        preferred_element_type=jnp.float32)
    acc = acc + b_ref[...]
    if relu:
        acc = jnp.maximum(acc, 0.0)
    if n_out == 1:
        outs[0][...] = acc
    else:
        w = acc.shape[1] // n_out
        for k in range(n_out):
            outs[k][...] = acc[:, k * w:(k + 1) * w]


def _dense_layer(aggs, d0, d1, xrs, wl, wr, bias, relu, n_out, mb=1000):
    m = xrs[0].shape[0]
    k = wl.shape[0]
    n = wl.shape[1]
    wa = aggs[0].shape[1]
    wx = xrs[0].shape[1]
    out_shape = [jax.ShapeDtypeStruct((m, n // n_out), jnp.float32)
                 for _ in range(n_out)]
    res = pl.pallas_call(
        functools.partial(_dense_body, len(aggs), len(xrs), n_out, relu),
        grid=(m // mb,),
        in_specs=(
            [pl.BlockSpec((mb, wa), lambda i: (i, 0)) for _ in aggs]
            + [pl.BlockSpec((mb, DEGW), lambda i: (i, 0))] * 2
            + [pl.BlockSpec((mb, wx), lambda i: (i, 0)) for _ in xrs]
            + [pl.BlockSpec((k, n), lambda i: (0, 0)),
               pl.BlockSpec((k, n), lambda i: (0, 0)),
               pl.BlockSpec((1, n), lambda i: (0, 0))]
        ),
        out_specs=[pl.BlockSpec((mb, n // n_out), lambda i: (i, 0))
                   for _ in range(n_out)],
        out_shape=out_shape,
    )(*aggs, d0, d1, *xrs, wl, wr, bias)
    return res


def kernel(x, edge_index, W1_l, b1, W1_r, W2_l, b2, W2_r):
    src = edge_index[0].astype(jnp.int32)
    dst = edge_index[1].astype(jnp.int32)
    n_edges = src.shape[0]

    e_pad = N_SUBCORES * EDGES_PER_TILE - n_edges
    src_p = jnp.concatenate([src, jnp.zeros((e_pad,), jnp.int32)])
    dst_p = jnp.concatenate([dst, jnp.full((e_pad,), N_NODES, jnp.int32)])
    srcs = src_p.reshape(N_SUBCORES, N_CHUNKS, CHUNK)
    dsts = dst_p.reshape(N_SUBCORES, N_CHUNKS, CHUNK)

    zeros = jnp.zeros((ZROWS, WIDTH), jnp.float32)
    zeros_d = jnp.zeros((ZROWS, DEGW), jnp.float32)
    ones = jnp.ones((CHUNK, DEGW), jnp.float32)

    tabs1 = [x[:, k * WIDTH:(k + 1) * WIDTH] for k in range(4)]
    o0, o1, o2, o3, d0, d1 = _make_segsum(2, True)(
        *tabs1, srcs, dsts, zeros, ones, zeros_d)
    hs = _dense_layer([o0, o1, o2, o3], d0, d1, [x], W1_l, W1_r,
                      b1.reshape(1, -1), relu=True, n_out=8)

    a2 = _make_segsum(4, False, nbuf=3)(*hs, srcs, dsts, zeros)
    out, = _dense_layer(list(a2), d0, d1, hs, W2_l, W2_r, b2.reshape(1, -1),
                        relu=False, n_out=1)
    return out
